# baseline (speedup 1.0000x reference)
import functools

import jax
import jax.numpy as jnp
from jax import lax
from jax.experimental import pallas as pl
from jax.experimental.pallas import tpu as pltpu
from jax.experimental.pallas import tpu_sc as plsc

_NC = 2
_NS = 16
_NW = _NC * _NS

_CHUNK = 16
_NBUF = 4


def _l2norm_rows(x_ref, o_ref):
    x = x_ref[...]
    n = jnp.sqrt(jnp.sum(x * x, axis=-1, keepdims=True))
    o_ref[...] = x / jnp.maximum(n, 1e-12)


def _normalize_table(table):
    v, d = table.shape
    blk = 512
    return pl.pallas_call(
        _l2norm_rows,
        out_shape=jax.ShapeDtypeStruct((v, d), table.dtype),
        grid=(v // blk,),
        in_specs=[pl.BlockSpec((blk, d), lambda i: (i, 0))],
        out_specs=pl.BlockSpec((blk, d), lambda i: (i, 0)),
    )(table)


def _sc_gather(table, idx):
    (b,) = idx.shape
    v, d = table.shape
    b_per_w = b // _NW
    nchunks = b_per_w // _CHUNK
    mesh = plsc.VectorSubcoreMesh(core_axis_name="c", subcore_axis_name="s")

    @functools.partial(
        pl.kernel,
        mesh=mesh,
        out_type=jax.ShapeDtypeStruct((b, d), table.dtype),
        scratch_types=(
            [pltpu.VMEM((b_per_w,), jnp.int32)]
            + [pltpu.VMEM((_CHUNK, d), jnp.float32) for _ in range(_NBUF)]
            + [pltpu.SemaphoreType.DMA for _ in range(_NBUF)]
        ),
    )
    def gather_kernel(table_hbm, idx_hbm, out_hbm, idx_v, *bufs_sems):
        bufs = bufs_sems[:_NBUF]
        gsems = bufs_sems[_NBUF:]
        wid = lax.axis_index("s") * _NC + lax.axis_index("c")
        base = wid * b_per_w
        pltpu.sync_copy(idx_hbm.at[pl.ds(base, b_per_w)], idx_v)

        def start_gather(chunk, b):
            pltpu.async_copy(
                table_hbm.at[idx_v.at[pl.ds(chunk * _CHUNK, _CHUNK)]],
                bufs[b], gsems[b],
            )

        def wait_gather(chunk, b):
            pltpu.make_async_copy(
                table_hbm.at[idx_v.at[pl.ds(chunk * _CHUNK, _CHUNK)]],
                bufs[b], gsems[b],
            ).wait()

        def write_out(chunk, b):
            pltpu.sync_copy(
                bufs[b], out_hbm.at[pl.ds(base + chunk * _CHUNK, _CHUNK)]
            )

        for b in range(_NBUF):
            start_gather(b, b)

        @pl.loop(0, nchunks - _NBUF, step=_NBUF)
        def _(j):
            for b in range(_NBUF):
                chunk = j + b
                wait_gather(chunk, b)
                write_out(chunk, b)
                start_gather(chunk + _NBUF, b)

        for b in range(_NBUF):
            chunk = nchunks - _NBUF + b
            wait_gather(chunk, b)
            write_out(chunk, b)

    return gather_kernel(table, idx)


def kernel(positions, pos_embeds):
    d = pos_embeds.shape[1]
    table_n = _normalize_table(pos_embeds)
    out = _sc_gather(table_n, positions.reshape(-1))
    return out.reshape(positions.shape + (d,))


# speedup vs baseline: 1.0323x; 1.0323x over previous
import functools

import jax
import jax.numpy as jnp
from jax import lax
from jax.experimental import pallas as pl
from jax.experimental.pallas import tpu as pltpu
from jax.experimental.pallas import tpu_sc as plsc

_NC = 2
_NS = 16
_NW = _NC * _NS

_CHUNK = 16
_NBUF = 4


def _l2norm_rows(x_ref, o_ref):
    x = x_ref[...]
    n = jnp.sqrt(jnp.sum(x * x, axis=-1, keepdims=True))
    o_ref[...] = x / jnp.maximum(n, 1e-12)


def _normalize_table(table):
    v, d = table.shape
    blk = 2048
    return pl.pallas_call(
        _l2norm_rows,
        out_shape=jax.ShapeDtypeStruct((v, d), table.dtype),
        grid=(v // blk,),
        in_specs=[pl.BlockSpec((blk, d), lambda i: (i, 0))],
        out_specs=pl.BlockSpec((blk, d), lambda i: (i, 0)),
    )(table)


def _sc_gather(table, idx):
    (b,) = idx.shape
    v, d = table.shape
    b_per_w = b // _NW
    nchunks = b_per_w // _CHUNK
    mesh = plsc.VectorSubcoreMesh(core_axis_name="c", subcore_axis_name="s")

    @functools.partial(
        pl.kernel,
        mesh=mesh,
        out_type=jax.ShapeDtypeStruct((b, d), table.dtype),
        scratch_types=(
            [pltpu.VMEM((b_per_w,), jnp.int32)]
            + [pltpu.VMEM((_CHUNK, d), jnp.float32) for _ in range(_NBUF)]
            + [pltpu.SemaphoreType.DMA for _ in range(_NBUF)]
        ),
    )
    def gather_kernel(table_hbm, idx_hbm, out_hbm, idx_v, *bufs_sems):
        bufs = bufs_sems[:_NBUF]
        gsems = bufs_sems[_NBUF:]
        wid = lax.axis_index("s") * _NC + lax.axis_index("c")
        base = wid * b_per_w
        pltpu.sync_copy(idx_hbm.at[pl.ds(base, b_per_w)], idx_v)

        def start_gather(chunk, b):
            pltpu.async_copy(
                table_hbm.at[idx_v.at[pl.ds(chunk * _CHUNK, _CHUNK)]],
                bufs[b], gsems[b],
            )

        def wait_gather(chunk, b):
            pltpu.make_async_copy(
                table_hbm.at[idx_v.at[pl.ds(chunk * _CHUNK, _CHUNK)]],
                bufs[b], gsems[b],
            ).wait()

        def write_out(chunk, b):
            pltpu.sync_copy(
                bufs[b], out_hbm.at[pl.ds(base + chunk * _CHUNK, _CHUNK)]
            )

        for b in range(_NBUF):
            start_gather(b, b)

        @pl.loop(0, nchunks - _NBUF, step=_NBUF)
        def _(j):
            for b in range(_NBUF):
                chunk = j + b
                wait_gather(chunk, b)
                write_out(chunk, b)
                start_gather(chunk + _NBUF, b)

        for b in range(_NBUF):
            chunk = nchunks - _NBUF + b
            wait_gather(chunk, b)
            write_out(chunk, b)

    return gather_kernel(table, idx)


def kernel(positions, pos_embeds):
    d = pos_embeds.shape[1]
    table_n = _normalize_table(pos_embeds)
    out = _sc_gather(table_n, positions.reshape(-1))
    return out.reshape(positions.shape + (d,))
